# baseline (speedup 1.0000x reference)
import functools

import jax
import jax.numpy as jnp
from jax import lax
from jax.experimental import pallas as pl
from jax.experimental.pallas import tpu as pltpu
from jax.experimental.pallas import tpu_sc as plsc

BATCH = 256
MAXLEN = 50
CLIP = 30
HALF = CLIP // 2
TCHUNK = 5
NCHUNK = HALF // TCHUNK
DIM = 128
LANES = 16
NVEC = DIM // LANES
PAIR_ROWS = 16
OUT_ROWS = 8


def _recip_vec(den_f32_scalar):
    nf = jnp.broadcast_to(den_f32_scalar, (LANES,))
    seed = jnp.asarray(0x7EF311C3, jnp.int32) - lax.bitcast_convert_type(
        nf, jnp.int32
    )
    y = lax.bitcast_convert_type(seed, jnp.float32)
    two = jnp.full((LANES,), 2.0, jnp.float32)
    y = y * (two - nf * y)
    y = y * (two - nf * y)
    y = y * (two - nf * y)
    return y


def _make_kernel():
    info = plsc.get_sparse_core_info()
    nc, ns = info.num_cores, info.num_subcores
    rows_per_sc = BATCH // nc

    mesh = plsc.VectorSubcoreMesh(core_axis_name="c", subcore_axis_name="s")

    @functools.partial(
        pl.kernel,
        mesh=mesh,
        out_type=jax.ShapeDtypeStruct((BATCH, DIM), jnp.float32),
        scratch_types=[
            pltpu.VMEM((BATCH + LANES,), jnp.int32),
            pltpu.VMEM((HALF, PAIR_ROWS, DIM), jnp.float32),
            pltpu.VMEM((PAIR_ROWS, DIM), jnp.float32),
            pltpu.VMEM((OUT_ROWS, DIM), jnp.float32),
            pltpu.VMEM_SHARED((ns, PAIR_ROWS, DIM), jnp.float32),
            pltpu.SemaphoreType.DMA,
        ],
    )
    def seq_mean(xt_hbm, len_hbm, out_hbm, len_v, buf_v, part_v, out_v,
                 shared_v, sem):
        c = lax.axis_index("c")
        s = lax.axis_index("s")
        half = s % 2
        wlo = half * HALF
        pair_base = c * rows_per_sc + (s // 2) * PAIR_ROWS
        base8 = pair_base + (s % 2) * OUT_ROWS
        roff = (s % 2) * OUT_ROWS

        pltpu.sync_copy(len_hbm, len_v.at[pl.ds(0, BATCH)])
        copies = [
            pltpu.async_copy(
                xt_hbm.at[
                    pl.ds(wlo + ch * TCHUNK, TCHUNK),
                    pl.ds(pair_base, PAIR_ROWS),
                ],
                buf_v.at[pl.ds(ch * TCHUNK, TCHUNK)],
                sem,
            )
            for ch in range(NCHUNK)
        ]

        zeros = tuple(jnp.zeros((LANES,), jnp.float32) for _ in range(NVEC))

        for ch in range(NCHUNK):
            copies[ch].wait()
            lo = ch * TCHUNK

            def row_body(r, _, _lo=lo):
                ln = len_v[pl.ds(pair_base + r, LANES)][0]
                lnc = jnp.minimum(ln, CLIP)
                hi_local = jnp.clip(lnc - wlo, _lo, _lo + TCHUNK)
                if _lo == 0:
                    accs = zeros
                else:
                    accs = tuple(
                        part_v[r, pl.ds(k * LANES, LANES)]
                        for k in range(NVEC)
                    )

                def t_body(t, a):
                    return tuple(
                        ak + buf_v[t, r, pl.ds(k * LANES, LANES)]
                        for k, ak in enumerate(a)
                    )

                accs = lax.fori_loop(_lo, hi_local, t_body, accs)
                for k in range(NVEC):
                    part_v[r, pl.ds(k * LANES, LANES)] = accs[k]
                return 0

            lax.fori_loop(0, PAIR_ROWS, row_body, 0)

        pltpu.sync_copy(part_v, shared_v.at[s])
        plsc.subcore_barrier()
        nbr = jnp.where(half == 0, s + 1, s - 1)
        pltpu.sync_copy(shared_v.at[nbr, pl.ds(roff, OUT_ROWS)], out_v)

        def fin_body(r, _):
            ln = len_v[pl.ds(base8 + r, LANES)][0]
            lnc = jnp.minimum(ln, CLIP)
            den = jnp.maximum(lnc, 1).astype(jnp.float32)
            scale = _recip_vec(den)
            for k in range(NVEC):
                sl = pl.ds(k * LANES, LANES)
                out_v[r, sl] = (out_v[r, sl] + part_v[roff + r, sl]) * scale
            return 0

        lax.fori_loop(0, OUT_ROWS, fin_body, 0)

        pltpu.sync_copy(out_v, out_hbm.at[pl.ds(base8, OUT_ROWS)])

    return seq_mean


_seq_mean = _make_kernel()


def kernel(opt_seq_embedding, length):
    xt = jnp.transpose(opt_seq_embedding, (1, 0, 2))
    return _seq_mean(xt, length)


# speedup vs baseline: 1.0929x; 1.0929x over previous
import functools

import jax
import jax.numpy as jnp
from jax import lax
from jax.experimental import pallas as pl
from jax.experimental.pallas import tpu as pltpu
from jax.experimental.pallas import tpu_sc as plsc

BATCH = 256
MAXLEN = 50
CLIP = 30
COPYLEN = 30
TCHUNK = 15
NCHUNK = COPYLEN // TCHUNK
DIM = 128
LANES = 16
NVEC = DIM // LANES


def _recip_vec(den_f32_scalar):
    nf = jnp.broadcast_to(den_f32_scalar, (LANES,))
    seed = jnp.asarray(0x7EF311C3, jnp.int32) - lax.bitcast_convert_type(
        nf, jnp.int32
    )
    y = lax.bitcast_convert_type(seed, jnp.float32)
    two = jnp.full((LANES,), 2.0, jnp.float32)
    y = y * (two - nf * y)
    y = y * (two - nf * y)
    y = y * (two - nf * y)
    return y


def _make_kernel():
    info = plsc.get_sparse_core_info()
    nc, ns = info.num_cores, info.num_subcores
    nw = nc * ns
    rows_per_w = BATCH // nw

    mesh = plsc.VectorSubcoreMesh(core_axis_name="c", subcore_axis_name="s")

    @functools.partial(
        pl.kernel,
        mesh=mesh,
        out_type=jax.ShapeDtypeStruct((BATCH, DIM), jnp.float32),
        scratch_types=[
            pltpu.VMEM((BATCH + LANES,), jnp.int32),
            pltpu.VMEM((COPYLEN, 8, DIM), jnp.float32),
            pltpu.VMEM((rows_per_w, DIM), jnp.float32),
            pltpu.SemaphoreType.DMA,
        ],
    )
    def seq_mean(xt_hbm, len_hbm, out_hbm, len_v, buf_v, out_v, sem):
        wid = lax.axis_index("s") * nc + lax.axis_index("c")
        base = wid * rows_per_w

        pltpu.sync_copy(len_hbm, len_v.at[pl.ds(0, BATCH)])
        copies = [
            pltpu.async_copy(
                xt_hbm.at[pl.ds(c * TCHUNK, TCHUNK), pl.ds(base, rows_per_w)],
                buf_v.at[pl.ds(c * TCHUNK, TCHUNK)],
                sem,
            )
            for c in range(NCHUNK)
        ]

        zeros = tuple(jnp.zeros((LANES,), jnp.float32) for _ in range(NVEC))

        for c in range(NCHUNK):
            copies[c].wait()
            lo = c * TCHUNK

            def row_body(r, _, _lo=lo, _last=(c == NCHUNK - 1)):
                ln = len_v[pl.ds(base + r, LANES)][0]
                lnc = jnp.minimum(ln, CLIP)
                hi = jnp.maximum(jnp.minimum(lnc, _lo + TCHUNK), _lo)
                if _lo == 0:
                    accs = zeros
                else:
                    accs = tuple(
                        out_v[r, pl.ds(k * LANES, LANES)] for k in range(NVEC)
                    )

                def t_body(t, a):
                    return tuple(
                        ak + buf_v[t, r, pl.ds(k * LANES, LANES)]
                        for k, ak in enumerate(a)
                    )

                accs = lax.fori_loop(_lo, hi, t_body, accs)
                if _last:
                    den = jnp.maximum(lnc, 1).astype(jnp.float32)
                    scale = _recip_vec(den)
                    accs = tuple(ak * scale for ak in accs)
                for k in range(NVEC):
                    out_v[r, pl.ds(k * LANES, LANES)] = accs[k]
                return 0

            lax.fori_loop(0, rows_per_w, row_body, 0)

        pltpu.sync_copy(out_v, out_hbm.at[pl.ds(base, rows_per_w)])

    return seq_mean


_seq_mean = _make_kernel()


def kernel(opt_seq_embedding, length):
    xt = jnp.transpose(opt_seq_embedding, (1, 0, 2))
    return _seq_mean(xt, length)
